# baseline (speedup 1.0000x reference)
import jax
import jax.numpy as jnp
import numpy as np
from jax import lax
from jax.experimental import pallas as pl
from jax.experimental.pallas import tpu as pltpu
from jax.experimental.pallas import tpu_sc as plsc

N = 10000
N2 = 10240
NREL = 200
NRELP = 256
DIN = 32
DHID = 32
T = 3
E = 160000
EPS = 1e-6
B = 4

CH = 640
NCH = E // CH
VPC = CH // 16
NV2 = N2 // 16
NB = 2048
NBLK = N2 // NB

_f32 = jnp.float32


def _sc_agg_body(x_hbm, epk_hbm, relt_hbm,
                 s_out, sq_out, mx_out, mn_out, deg_out,
                 xc0, xc1, as0, aq0, amx0, amn0, as1, aq1, amx1, amn1,
                 adeg, tmpi, relv, eb0, eb1, sem0, sem1):
    cid = lax.axis_index("c")
    sid = lax.axis_index("s")
    wid = sid * 2 + cid
    d = wid

    pltpu.sync_copy(relt_hbm.at[d], relv)

    ones16 = jnp.ones((16,), _f32)
    zero16 = jnp.zeros((16,), _f32)
    ninf16 = jnp.full((16,), -jnp.inf, _f32)
    pinf16 = jnp.full((16,), jnp.inf, _f32)
    lane16 = lax.iota(jnp.int32, 16)
    true16 = jnp.ones((16,), jnp.bool_)

    def rmw_maxmin_dup(accx, accn, idx, val):
        def cond(c):
            return jnp.any(jnp.logical_or(c[0], c[1]))

        def body(c):
            ax, an = c
            gx = plsc.load_gather(accx, [idx])
            wx = jnp.logical_and(ax, val > gx)
            plsc.store_scatter(accx, [idx], val, mask=wx)
            gn = plsc.load_gather(accn, [idx])
            wn = jnp.logical_and(an, val < gn)
            plsc.store_scatter(accn, [idx], val, mask=wn)
            gx2 = plsc.load_gather(accx, [idx])
            gn2 = plsc.load_gather(accn, [idx])
            return (jnp.logical_and(wx, gx2 < val),
                    jnp.logical_and(wn, gn2 > val))

        lax.while_loop(cond, body, (true16, true16))

    def rmw_maxmin_nodup(accx, accn, idx, val):
        gx = plsc.load_gather(accx, [idx])
        plsc.store_scatter(accx, [idx], jnp.maximum(gx, val))
        gn = plsc.load_gather(accn, [idx])
        plsc.store_scatter(accn, [idx], jnp.minimum(gn, val))

    for p in range(2):
        b0 = 2 * p
        xcs = (xc0, xc1)
        accs = ((as0, aq0, amx0, amn0), (as1, aq1, amx1, amn1))

        def initv(i, carry):
            sl = pl.ds(i * 16, 16)
            for q in range(2):
                accs[q][0][sl] = zero16
                accs[q][1][sl] = zero16
                accs[q][2][sl] = ninf16
                accs[q][3][sl] = pinf16
            if p == 0:
                adeg[sl] = zero16
            return carry

        lax.fori_loop(0, NV2, initv, 0)

        pltpu.sync_copy(x_hbm.at[b0 * 32 + d], xc0)
        pltpu.sync_copy(x_hbm.at[(b0 + 1) * 32 + d], xc1)

        do_deg = (p == 0)

        def start(ci, buf, sem):
            pltpu.make_async_copy(
                epk_hbm.at[pl.ds(ci * (3 * CH), 3 * CH)], buf, sem).start()

        def wait(buf, sem):
            pltpu.make_async_copy(
                epk_hbm.at[pl.ds(0, 3 * CH)], buf, sem).wait()

        def process(buf):
            def one(vi):
                sv = buf[pl.ds(vi * 16, 16)]
                dv = buf[pl.ds(CH + vi * 16, 16)]
                tv = buf[pl.ds(2 * CH + vi * 16, 16)]
                rv = plsc.load_gather(relv, [tv])
                if do_deg:

                    @pl.when(wid == 0)
                    def _():
                        plsc.addupdate_scatter(adeg, [dv], ones16)

                plsc.store_scatter(tmpi, [dv], lane16)
                gl = plsc.load_gather(tmpi, [dv])
                nodup = jnp.all(gl == lane16)

                msgs = []
                for q in range(2):
                    xv = plsc.load_gather(xcs[q], [sv])
                    msg = xv * rv
                    msgs.append(msg)
                    plsc.addupdate_scatter(accs[q][0], [dv], msg)
                    plsc.addupdate_scatter(accs[q][1], [dv], msg * msg)

                for q in range(2):
                    rmw_maxmin_nodup(accs[q][2], accs[q][3], dv, msgs[q])
                return nodup, dv, msgs

            def vec(vi, c2):
                res = [one(4 * vi + u) for u in range(4)]
                allnodup = res[0][0]
                for u in range(1, 4):
                    allnodup = jnp.logical_and(allnodup, res[u][0])

                @pl.when(jnp.logical_not(allnodup))
                def _():
                    for _, dv, msgs in res:
                        for q in range(2):
                            rmw_maxmin_dup(accs[q][2], accs[q][3], dv, msgs[q])

                return c2

            lax.fori_loop(0, VPC // 4, vec, 0)

        start(0, eb0, sem0)

        def loop(i, carry):
            c1 = 2 * i + 1
            wait(eb0, sem0)
            start(c1, eb1, sem1)
            process(eb0)
            wait(eb1, sem1)

            @pl.when(c1 + 1 < NCH)
            def _():
                start(c1 + 1, eb0, sem0)

            process(eb1)
            return carry

        lax.fori_loop(0, NCH // 2, loop, 0)

        pltpu.sync_copy(as0, s_out.at[b0 * 32 + d])
        pltpu.sync_copy(as1, s_out.at[(b0 + 1) * 32 + d])
        pltpu.sync_copy(aq0, sq_out.at[b0 * 32 + d])
        pltpu.sync_copy(aq1, sq_out.at[(b0 + 1) * 32 + d])
        pltpu.sync_copy(amx0, mx_out.at[b0 * 32 + d])
        pltpu.sync_copy(amx1, mx_out.at[(b0 + 1) * 32 + d])
        pltpu.sync_copy(amn0, mn_out.at[b0 * 32 + d])
        pltpu.sync_copy(amn1, mn_out.at[(b0 + 1) * 32 + d])
        if p == 0:

            @pl.when(wid == 0)
            def _():
                pltpu.sync_copy(adeg, deg_out)


def _make_sc_agg():
    mesh = plsc.VectorSubcoreMesh(core_axis_name="c", subcore_axis_name="s")
    out_type = (
        jax.ShapeDtypeStruct((B * DIN, N2), _f32),
        jax.ShapeDtypeStruct((B * DIN, N2), _f32),
        jax.ShapeDtypeStruct((B * DIN, N2), _f32),
        jax.ShapeDtypeStruct((B * DIN, N2), _f32),
        jax.ShapeDtypeStruct((N2,), _f32),
    )
    scratch = (
        [pltpu.VMEM((N2,), _f32) for _ in range(2)]
        + [pltpu.VMEM((N2,), _f32) for _ in range(8)]
        + [pltpu.VMEM((N2,), _f32)]
        + [pltpu.VMEM((N2,), jnp.int32)]
        + [pltpu.VMEM((NRELP,), _f32)]
        + [pltpu.VMEM((3 * CH,), jnp.int32) for _ in range(2)]
        + [pltpu.SemaphoreType.DMA, pltpu.SemaphoreType.DMA]
    )
    return pl.kernel(_sc_agg_body, mesh=mesh, out_type=out_type,
                     scratch_types=scratch,
                     compiler_params=pltpu.CompilerParams(
                         use_tc_tiling_on_sc=False,
                         needs_layout_passes=False))


_sc_agg = _make_sc_agg()


def _init_body(h0_ref, q_ref, out_ref):
    iot = lax.broadcasted_iota(jnp.int32, (DIN, N2), 1)
    for b in range(B):
        col = h0_ref[b, 0]
        qv = q_ref[b, :].reshape(DIN, 1)
        out_ref[b * DIN:(b + 1) * DIN, :] = jnp.where(
            iot == col, qv, jnp.zeros_like(qv))


def _run_init(h0, query):
    return pl.pallas_call(
        _init_body,
        out_shape=jax.ShapeDtypeStruct((B * DIN, N2), _f32),
    )(h0, query)


def _combine_body(s_ref, sq_ref, mx_ref, mn_ref, x_ref, init_ref, deg_ref,
                  wpt_ref, b0_ref, out_ref, xnext_ref):
    j = pl.program_id(0)
    degf = deg_ref[...] + 1.0
    meanlog = jnp.sum(jnp.log(degf)) / N
    degb = deg_ref[:, pl.ds(j * NB, NB)] + 1.0
    scl = jnp.log(degb) / meanlog
    iscl = 1.0 / jnp.maximum(scl, 1e-2)

    colid = lax.broadcasted_iota(jnp.int32, (1, NB), 1) + j * NB
    valid = colid < N

    x = x_ref[...]
    s = s_ref[...] + x
    sq = sq_ref[...] + x * x
    mx = jnp.maximum(mx_ref[...], x)
    mn = jnp.minimum(mn_ref[...], x)
    mean = s / degb
    std = jnp.sqrt(jnp.maximum(sq / degb - mean * mean, EPS))
    wpt = wpt_ref[...]
    bias = b0_ref[...]

    for b in range(B):
        r = slice(b * DIN, (b + 1) * DIN)
        pieces = [x[r]]
        for stat in (mean[r], mx[r], mn[r], std[r]):
            pieces.append(stat)
            pieces.append(stat * scl)
            pieces.append(stat * iscl)
        A = jnp.concatenate(pieces, axis=0)
        o = jnp.dot(wpt, A, preferred_element_type=_f32) + bias
        o = jnp.maximum(o, 0.0)
        o = jnp.where(valid, o, 0.0)
        out_ref[r, :] = o
        xnext_ref[r, :] = 0.5 * o + 0.5 * init_ref[r, :]


def _run_combine(s, sq, mx, mn, x, init_t, deg, wpt, b0c):
    big = pl.BlockSpec((B * DIN, NB), lambda j: (0, j))
    full_deg = pl.BlockSpec((1, N2), lambda j: (0, 0))
    full_w = pl.BlockSpec((DIN, 13 * DIN), lambda j: (0, 0))
    full_b = pl.BlockSpec((DIN, 1), lambda j: (0, 0))
    return pl.pallas_call(
        _combine_body,
        grid=(NBLK,),
        in_specs=[big, big, big, big, big, big, full_deg, full_w, full_b],
        out_specs=[big, big],
        out_shape=[jax.ShapeDtypeStruct((B * DIN, N2), _f32),
                   jax.ShapeDtypeStruct((B * DIN, N2), _f32)],
    )(s, sq, mx, mn, x, init_t, deg, wpt, b0c)


def _score_body(feat_ref, q_ref, t_ref, w1t_ref, b1_ref, w2t_ref, b2_ref,
                out_ref):
    nneg = t_ref.shape[1]
    iot = lax.broadcasted_iota(jnp.int32, (N2, nneg), 0)
    for b in range(B):
        t = t_ref[b, :]
        onehot = (iot == t[None, :]).astype(_f32)
        G = jnp.dot(feat_ref[b * DIN:(b + 1) * DIN, :], onehot,
                    preferred_element_type=_f32)
        qcol = q_ref[b, :].reshape(DIN, 1)
        F = jnp.concatenate([G, jnp.broadcast_to(qcol, (DIN, nneg))], axis=0)
        H = jnp.dot(w1t_ref[...], F, preferred_element_type=_f32) + b1_ref[...]
        H = jnp.maximum(H, 0.0)
        sc = jnp.dot(w2t_ref[...], H, preferred_element_type=_f32) + b2_ref[...]
        out_ref[b:b + 1, :] = sc


def _run_score(feat, query, t_new, w1t, b1c, w2t, b2c):
    nneg = t_new.shape[1]
    return pl.pallas_call(
        _score_body,
        out_shape=jax.ShapeDtypeStruct((B, nneg), _f32),
    )(feat, query, t_new, w1t, b1c, w2t, b2c)


_PERM = np.concatenate(
    [np.arange(DIN)]
    + [DIN + (np.arange(DIN) * 4 + k) * 3 + j for k in range(4) for j in range(3)]
)


def kernel(query_emb, rel0, W0, b0, mlp_W1, mlp_b1, mlp_W2, mlp_b2,
           edge_index, edge_type, query_triple):
    h, r, t = query_triple[..., 0], query_triple[..., 1], query_triple[..., 2]
    is_t_neg = jnp.all(h == h[:, :1], axis=-1, keepdims=True)
    t_new = jnp.where(is_t_neg, t, h).astype(jnp.int32)
    r_new = jnp.where(is_t_neg, r, r + NREL // 2)
    h0 = jnp.where(is_t_neg, h, t)[:, 0].astype(jnp.int32)
    query = query_emb[r_new[:, 0]]

    wpt = jnp.asarray(W0)[_PERM].T
    b0c = b0.reshape(DIN, 1)
    relt = jnp.pad(rel0.T, ((0, 0), (0, NRELP - NREL)))
    w1t = mlp_W1.T
    b1c = mlp_b1.reshape(DIN + DHID, 1)
    w2t = mlp_W2.T
    b2c = mlp_b2.reshape(1, 1)

    ei = edge_index.astype(jnp.int32)
    et = edge_type.astype(jnp.int32)

    def pack_edges(src, dst, etype):
        a = jnp.stack([src, dst, etype], axis=0)
        a = a.reshape(3, NCH, CH).transpose(1, 0, 2)
        return a.reshape(NCH * 3 * CH)

    init_t = _run_init(h0.reshape(B, 1), query)
    x = init_t
    out = None
    for l in range(T):
        epk = pack_edges(ei[l, 0], ei[l, 1], et[l])
        s, sq, mx, mn, deg = _sc_agg(x, epk, relt)
        out, xnext = _run_combine(s, sq, mx, mn, x, init_t, deg.reshape(1, N2),
                                  wpt, b0c)
        x = xnext

    score = _run_score(out, query, t_new, w1t, b1c, w2t, b2c)
    return score


# speedup vs baseline: 1.0677x; 1.0677x over previous
import jax
import jax.numpy as jnp
import numpy as np
from jax import lax
from jax.experimental import pallas as pl
from jax.experimental.pallas import tpu as pltpu
from jax.experimental.pallas import tpu_sc as plsc

N = 10000
N2 = 10240
NREL = 200
NRELP = 256
DIN = 32
DHID = 32
T = 3
E = 160000
EPS = 1e-6
B = 4

CH = 640
NCH = E // CH
VPC = CH // 16
NV2 = N2 // 16
NB = 2048
NBLK = N2 // NB

_f32 = jnp.float32


def _sc_agg_body(x_hbm, epk_hbm, relt_hbm,
                 s_out, sq_out, mx_out, mn_out, deg_out,
                 xc0, xc1, as0, aq0, amx0, amn0, as1, aq1, amx1, amn1,
                 adeg, tmpi, relv, eb0, eb1, sem0, sem1):
    cid = lax.axis_index("c")
    sid = lax.axis_index("s")
    wid = sid * 2 + cid
    d = wid

    pltpu.sync_copy(relt_hbm.at[d], relv)

    ones16 = jnp.ones((16,), _f32)
    zero16 = jnp.zeros((16,), _f32)
    ninf16 = jnp.full((16,), -jnp.inf, _f32)
    pinf16 = jnp.full((16,), jnp.inf, _f32)
    lane16 = lax.iota(jnp.int32, 16)
    true16 = jnp.ones((16,), jnp.bool_)

    def rmw_maxmin_dup(accx, accn, idx, val):
        def cond(c):
            return jnp.any(jnp.logical_or(c[0], c[1]))

        def body(c):
            ax, an = c
            gx = plsc.load_gather(accx, [idx])
            wx = jnp.logical_and(ax, val > gx)
            plsc.store_scatter(accx, [idx], val, mask=wx)
            gn = plsc.load_gather(accn, [idx])
            wn = jnp.logical_and(an, val < gn)
            plsc.store_scatter(accn, [idx], val, mask=wn)
            gx2 = plsc.load_gather(accx, [idx])
            gn2 = plsc.load_gather(accn, [idx])
            return (jnp.logical_and(wx, gx2 < val),
                    jnp.logical_and(wn, gn2 > val))

        lax.while_loop(cond, body, (true16, true16))

    def rmw_maxmin_nodup(accx, accn, idx, val):
        gx = plsc.load_gather(accx, [idx])
        plsc.store_scatter(accx, [idx], jnp.maximum(gx, val))
        gn = plsc.load_gather(accn, [idx])
        plsc.store_scatter(accn, [idx], jnp.minimum(gn, val))

    for p in range(2):
        b0 = 2 * p
        xcs = (xc0, xc1)
        accs = ((as0, aq0, amx0, amn0), (as1, aq1, amx1, amn1))

        def initv(i, carry):
            sl = pl.ds(i * 16, 16)
            for q in range(2):
                accs[q][0][sl] = zero16
                accs[q][1][sl] = zero16
                accs[q][2][sl] = ninf16
                accs[q][3][sl] = pinf16
            if p == 0:
                adeg[sl] = zero16
            return carry

        lax.fori_loop(0, NV2, initv, 0)

        pltpu.sync_copy(x_hbm.at[b0 * 32 + d], xc0)
        pltpu.sync_copy(x_hbm.at[(b0 + 1) * 32 + d], xc1)

        do_deg = (p == 0)

        def start(ci, buf, sem):
            pltpu.make_async_copy(
                epk_hbm.at[pl.ds(ci * (3 * CH), 3 * CH)], buf, sem).start()

        def wait(buf, sem):
            pltpu.make_async_copy(
                epk_hbm.at[pl.ds(0, 3 * CH)], buf, sem).wait()

        def process(buf):
            def one(vi):
                sv = buf[pl.ds(vi * 16, 16)]
                dv = buf[pl.ds(CH + vi * 16, 16)]
                tv = buf[pl.ds(2 * CH + vi * 16, 16)]
                rv = plsc.load_gather(relv, [tv])
                if do_deg:

                    @pl.when(wid == 0)
                    def _():
                        plsc.addupdate_scatter(adeg, [dv], ones16)

                plsc.store_scatter(tmpi, [dv], lane16)
                gl = plsc.load_gather(tmpi, [dv])
                nodup = jnp.all(gl == lane16)

                msgs = []
                for q in range(2):
                    xv = plsc.load_gather(xcs[q], [sv])
                    msg = xv * rv
                    msgs.append(msg)
                    plsc.addupdate_scatter(accs[q][0], [dv], msg)
                    plsc.addupdate_scatter(accs[q][1], [dv], msg * msg)

                for q in range(2):
                    rmw_maxmin_nodup(accs[q][2], accs[q][3], dv, msgs[q])

                @pl.when(jnp.logical_not(nodup))
                def _():
                    for q in range(2):
                        rmw_maxmin_dup(accs[q][2], accs[q][3], dv, msgs[q])

            def vec(vi, c2):
                for u in range(8):
                    one(8 * vi + u)
                return c2

            lax.fori_loop(0, VPC // 8, vec, 0)

        start(0, eb0, sem0)

        def loop(i, carry):
            c1 = 2 * i + 1
            wait(eb0, sem0)
            start(c1, eb1, sem1)
            process(eb0)
            wait(eb1, sem1)

            @pl.when(c1 + 1 < NCH)
            def _():
                start(c1 + 1, eb0, sem0)

            process(eb1)
            return carry

        lax.fori_loop(0, NCH // 2, loop, 0)

        pltpu.sync_copy(as0, s_out.at[b0 * 32 + d])
        pltpu.sync_copy(as1, s_out.at[(b0 + 1) * 32 + d])
        pltpu.sync_copy(aq0, sq_out.at[b0 * 32 + d])
        pltpu.sync_copy(aq1, sq_out.at[(b0 + 1) * 32 + d])
        pltpu.sync_copy(amx0, mx_out.at[b0 * 32 + d])
        pltpu.sync_copy(amx1, mx_out.at[(b0 + 1) * 32 + d])
        pltpu.sync_copy(amn0, mn_out.at[b0 * 32 + d])
        pltpu.sync_copy(amn1, mn_out.at[(b0 + 1) * 32 + d])
        if p == 0:

            @pl.when(wid == 0)
            def _():
                pltpu.sync_copy(adeg, deg_out)


def _make_sc_agg():
    mesh = plsc.VectorSubcoreMesh(core_axis_name="c", subcore_axis_name="s")
    out_type = (
        jax.ShapeDtypeStruct((B * DIN, N2), _f32),
        jax.ShapeDtypeStruct((B * DIN, N2), _f32),
        jax.ShapeDtypeStruct((B * DIN, N2), _f32),
        jax.ShapeDtypeStruct((B * DIN, N2), _f32),
        jax.ShapeDtypeStruct((N2,), _f32),
    )
    scratch = (
        [pltpu.VMEM((N2,), _f32) for _ in range(2)]
        + [pltpu.VMEM((N2,), _f32) for _ in range(8)]
        + [pltpu.VMEM((N2,), _f32)]
        + [pltpu.VMEM((N2,), jnp.int32)]
        + [pltpu.VMEM((NRELP,), _f32)]
        + [pltpu.VMEM((3 * CH,), jnp.int32) for _ in range(2)]
        + [pltpu.SemaphoreType.DMA, pltpu.SemaphoreType.DMA]
    )
    return pl.kernel(_sc_agg_body, mesh=mesh, out_type=out_type,
                     scratch_types=scratch,
                     compiler_params=pltpu.CompilerParams(
                         use_tc_tiling_on_sc=False,
                         needs_layout_passes=False))


_sc_agg = _make_sc_agg()


def _init_body(h0_ref, q_ref, out_ref):
    iot = lax.broadcasted_iota(jnp.int32, (DIN, N2), 1)
    for b in range(B):
        col = h0_ref[b, 0]
        qv = q_ref[b, :].reshape(DIN, 1)
        out_ref[b * DIN:(b + 1) * DIN, :] = jnp.where(
            iot == col, qv, jnp.zeros_like(qv))


def _run_init(h0, query):
    return pl.pallas_call(
        _init_body,
        out_shape=jax.ShapeDtypeStruct((B * DIN, N2), _f32),
    )(h0, query)


def _combine_body(s_ref, sq_ref, mx_ref, mn_ref, x_ref, init_ref, deg_ref,
                  wpt_ref, b0_ref, out_ref, xnext_ref):
    j = pl.program_id(0)
    degf = deg_ref[...] + 1.0
    meanlog = jnp.sum(jnp.log(degf)) / N
    degb = deg_ref[:, pl.ds(j * NB, NB)] + 1.0
    scl = jnp.log(degb) / meanlog
    iscl = 1.0 / jnp.maximum(scl, 1e-2)

    colid = lax.broadcasted_iota(jnp.int32, (1, NB), 1) + j * NB
    valid = colid < N

    x = x_ref[...]
    s = s_ref[...] + x
    sq = sq_ref[...] + x * x
    mx = jnp.maximum(mx_ref[...], x)
    mn = jnp.minimum(mn_ref[...], x)
    mean = s / degb
    std = jnp.sqrt(jnp.maximum(sq / degb - mean * mean, EPS))
    wpt = wpt_ref[...]
    bias = b0_ref[...]

    for b in range(B):
        r = slice(b * DIN, (b + 1) * DIN)
        pieces = [x[r]]
        for stat in (mean[r], mx[r], mn[r], std[r]):
            pieces.append(stat)
            pieces.append(stat * scl)
            pieces.append(stat * iscl)
        A = jnp.concatenate(pieces, axis=0)
        o = jnp.dot(wpt, A, preferred_element_type=_f32) + bias
        o = jnp.maximum(o, 0.0)
        o = jnp.where(valid, o, 0.0)
        out_ref[r, :] = o
        xnext_ref[r, :] = 0.5 * o + 0.5 * init_ref[r, :]


def _run_combine(s, sq, mx, mn, x, init_t, deg, wpt, b0c):
    big = pl.BlockSpec((B * DIN, NB), lambda j: (0, j))
    full_deg = pl.BlockSpec((1, N2), lambda j: (0, 0))
    full_w = pl.BlockSpec((DIN, 13 * DIN), lambda j: (0, 0))
    full_b = pl.BlockSpec((DIN, 1), lambda j: (0, 0))
    return pl.pallas_call(
        _combine_body,
        grid=(NBLK,),
        in_specs=[big, big, big, big, big, big, full_deg, full_w, full_b],
        out_specs=[big, big],
        out_shape=[jax.ShapeDtypeStruct((B * DIN, N2), _f32),
                   jax.ShapeDtypeStruct((B * DIN, N2), _f32)],
    )(s, sq, mx, mn, x, init_t, deg, wpt, b0c)


def _score_body(feat_ref, q_ref, t_ref, w1t_ref, b1_ref, w2t_ref, b2_ref,
                out_ref):
    nneg = t_ref.shape[1]
    iot = lax.broadcasted_iota(jnp.int32, (N2, nneg), 0)
    for b in range(B):
        t = t_ref[b, :]
        onehot = (iot == t[None, :]).astype(_f32)
        G = jnp.dot(feat_ref[b * DIN:(b + 1) * DIN, :], onehot,
                    preferred_element_type=_f32)
        qcol = q_ref[b, :].reshape(DIN, 1)
        F = jnp.concatenate([G, jnp.broadcast_to(qcol, (DIN, nneg))], axis=0)
        H = jnp.dot(w1t_ref[...], F, preferred_element_type=_f32) + b1_ref[...]
        H = jnp.maximum(H, 0.0)
        sc = jnp.dot(w2t_ref[...], H, preferred_element_type=_f32) + b2_ref[...]
        out_ref[b:b + 1, :] = sc


def _run_score(feat, query, t_new, w1t, b1c, w2t, b2c):
    nneg = t_new.shape[1]
    return pl.pallas_call(
        _score_body,
        out_shape=jax.ShapeDtypeStruct((B, nneg), _f32),
    )(feat, query, t_new, w1t, b1c, w2t, b2c)


_PERM = np.concatenate(
    [np.arange(DIN)]
    + [DIN + (np.arange(DIN) * 4 + k) * 3 + j for k in range(4) for j in range(3)]
)


def kernel(query_emb, rel0, W0, b0, mlp_W1, mlp_b1, mlp_W2, mlp_b2,
           edge_index, edge_type, query_triple):
    h, r, t = query_triple[..., 0], query_triple[..., 1], query_triple[..., 2]
    is_t_neg = jnp.all(h == h[:, :1], axis=-1, keepdims=True)
    t_new = jnp.where(is_t_neg, t, h).astype(jnp.int32)
    r_new = jnp.where(is_t_neg, r, r + NREL // 2)
    h0 = jnp.where(is_t_neg, h, t)[:, 0].astype(jnp.int32)
    query = query_emb[r_new[:, 0]]

    wpt = jnp.asarray(W0)[_PERM].T
    b0c = b0.reshape(DIN, 1)
    relt = jnp.pad(rel0.T, ((0, 0), (0, NRELP - NREL)))
    w1t = mlp_W1.T
    b1c = mlp_b1.reshape(DIN + DHID, 1)
    w2t = mlp_W2.T
    b2c = mlp_b2.reshape(1, 1)

    ei = edge_index.astype(jnp.int32)
    et = edge_type.astype(jnp.int32)

    def pack_edges(src, dst, etype):
        a = jnp.stack([src, dst, etype], axis=0)
        a = a.reshape(3, NCH, CH).transpose(1, 0, 2)
        return a.reshape(NCH * 3 * CH)

    init_t = _run_init(h0.reshape(B, 1), query)
    x = init_t
    out = None
    for l in range(T):
        epk = pack_edges(ei[l, 0], ei[l, 1], et[l])
        s, sq, mx, mn, deg = _sc_agg(x, epk, relt)
        out, xnext = _run_combine(s, sq, mx, mn, x, init_t, deg.reshape(1, N2),
                                  wpt, b0c)
        x = xnext

    score = _run_score(out, query, t_new, w1t, b1c, w2t, b2c)
    return score
